# baseline (speedup 1.0000x reference)
import jax
import jax.numpy as jnp
from jax.experimental import pallas as pl


def _pe_add_kernel(x_ref, e_ref, o_ref):
    o_ref[...] = x_ref[...] + e_ref[...][None, :, :]


def kernel(x, embeds):
    B, T, D = x.shape
    bt = 1024
    grid = (T // bt, B)
    return pl.pallas_call(
        _pe_add_kernel,
        grid=grid,
        in_specs=[
            pl.BlockSpec((1, bt, D), lambda t, b: (b, t, 0)),
            pl.BlockSpec((bt, D), lambda t, b: (t, 0)),
        ],
        out_specs=pl.BlockSpec((1, bt, D), lambda t, b: (b, t, 0)),
        out_shape=jax.ShapeDtypeStruct((B, T, D), x.dtype),
    )(x, embeds)


# speedup vs baseline: 1.0300x; 1.0300x over previous
import jax
import jax.numpy as jnp
from jax.experimental import pallas as pl


def _pe_add_kernel(x_ref, e_ref, o_ref):
    o_ref[...] = x_ref[...] + e_ref[...][None, :, :]


def kernel(x, embeds):
    B, T, D = x.shape
    bt = 256
    grid = (T // bt,)
    return pl.pallas_call(
        _pe_add_kernel,
        grid=grid,
        in_specs=[
            pl.BlockSpec((B, bt, D), lambda t: (0, t, 0)),
            pl.BlockSpec((bt, D), lambda t: (t, 0)),
        ],
        out_specs=pl.BlockSpec((B, bt, D), lambda t: (0, t, 0)),
        out_shape=jax.ShapeDtypeStruct((B, T, D), x.dtype),
    )(x, embeds)
